# Initial kernel scaffold; baseline (speedup 1.0000x reference)
import jax
import jax.numpy as jnp
from jax.experimental import pallas as pl


def kernel(x_obs, pos_obs, pos_query, W_enc1, b_enc1, W_enc2, b_enc2, W_gs0, W_gn0, b_g0, W_gs1, W_gn1, b_g1, W_q, b_q, W_k, b_k, W_v, b_v, w_rel, W_dec1, b_dec1, W_dec2, b_dec2):
    raise NotImplementedError("write your pallas kernel here")


# speedup vs baseline: 51.8032x; 51.8032x over previous
import functools

import jax
import jax.numpy as jnp
from jax import lax
from jax.experimental import pallas as pl
from jax.experimental.pallas import tpu as pltpu
from jax.experimental.pallas import tpu_sc as plsc

N_OBS = 10000
N_Q = 2048
LATENT = 128
K = 4
BIG = 3.0e38


def _enc_body(x_ref, w1_ref, b1_ref, w2_ref, b2_ref, o_ref):
    h = jnp.maximum(
        jnp.dot(x_ref[...], w1_ref[...], preferred_element_type=jnp.float32)
        + b1_ref[...],
        0.0,
    )
    o_ref[...] = (
        jnp.dot(h, w2_ref[...], preferred_element_type=jnp.float32) + b2_ref[...]
    )


def _encode(x_obs, W1, b1, W2, b2):
    return pl.pallas_call(
        _enc_body,
        out_shape=jax.ShapeDtypeStruct((N_OBS, LATENT), jnp.float32),
    )(x_obs, W1, b1, W2, b2)


_KNN_R = 400


def _knn_body(pos_ref, post_ref, nbr_ref):
    p = pos_ref[...]
    pt = post_ref[...]
    sqr = jnp.sum(p * p, axis=1, keepdims=True)
    sqc = jnp.sum(pt * pt, axis=0, keepdims=True)
    cross = jax.lax.dot_general(
        p, pt, (((1,), (0,)), ((), ())), preferred_element_type=jnp.float32
    )
    d2 = sqr + sqc - 2.0 * cross
    i = pl.program_id(0)
    rowid = jax.lax.broadcasted_iota(jnp.int32, (_KNN_R, N_OBS), 0) + i * _KNN_R
    colid = jax.lax.broadcasted_iota(jnp.int32, (_KNN_R, N_OBS), 1)
    d2 = jnp.where(colid == rowid, BIG, d2)
    cols = []
    for _ in range(K):
        m = jnp.min(d2, axis=1, keepdims=True)
        am = jnp.min(jnp.where(d2 <= m, colid, N_OBS), axis=1, keepdims=True)
        cols.append(am)
        d2 = jnp.where(colid == am, BIG, d2)
    nbr_ref[...] = jnp.concatenate(cols, axis=1)


def _knn(pos_obs, pos_obs_t):
    return pl.pallas_call(
        _knn_body,
        grid=(N_OBS // _KNN_R,),
        in_specs=[
            pl.BlockSpec((_KNN_R, 3), lambda i: (i, 0)),
            pl.BlockSpec((3, N_OBS), lambda i: (0, 0)),
        ],
        out_specs=pl.BlockSpec((_KNN_R, K), lambda i: (i, 0)),
        out_shape=jax.ShapeDtypeStruct((N_OBS, K), jnp.int32),
    )(pos_obs, pos_obs_t)


def _gnn_body(h_ref, agg_ref, ws_ref, wn_ref, b_ref, o_ref):
    h = h_ref[...]
    z = (
        jnp.dot(h, ws_ref[...], preferred_element_type=jnp.float32)
        + jnp.dot(agg_ref[...], wn_ref[...], preferred_element_type=jnp.float32)
        + b_ref[...]
    )
    o_ref[...] = h + jnp.maximum(z, 0.0)


def _gnn_layer(h, agg, Ws, Wn, b):
    return pl.pallas_call(
        _gnn_body,
        out_shape=jax.ShapeDtypeStruct((N_OBS, LATENT), jnp.float32),
    )(h, agg, Ws, Wn, b)


def _kv_body(h_ref, pos_ref, wkh_ref, wkp_ref, bk_ref, wvh_ref, wvp_ref, bv_ref,
             k_ref, v_ref):
    h = h_ref[...]
    p = pos_ref[...]
    k_ref[...] = (
        jnp.dot(h, wkh_ref[...], preferred_element_type=jnp.float32)
        + jnp.dot(p, wkp_ref[...], preferred_element_type=jnp.float32)
        + bk_ref[...]
    )
    v_ref[...] = (
        jnp.dot(h, wvh_ref[...], preferred_element_type=jnp.float32)
        + jnp.dot(p, wvp_ref[...], preferred_element_type=jnp.float32)
        + bv_ref[...]
    )


def _kv(h, pos_obs, Wkh, Wkp, bk, Wvh, Wvp, bv):
    return pl.pallas_call(
        _kv_body,
        out_shape=(
            jax.ShapeDtypeStruct((N_OBS, LATENT), jnp.float32),
            jax.ShapeDtypeStruct((N_OBS, LATENT), jnp.float32),
        ),
    )(h, pos_obs, Wkh, Wkp, bk, Wvh, Wvp, bv)


_ATT_Q = 256


def _attn_body(pq_ref, wq_ref, bq_ref, k_ref, v_ref, post_ref, wrel_ref,
               wd1_ref, bd1_ref, wd2_ref, bd2_ref, o_ref):
    q = (
        jnp.dot(pq_ref[...], wq_ref[...], preferred_element_type=jnp.float32)
        + bq_ref[...]
    )
    logits = jax.lax.dot_general(
        q, k_ref[...], (((1,), (1,)), ((), ())), preferred_element_type=jnp.float32
    ) * jnp.float32(1.0 / (LATENT ** 0.5))
    bo = jnp.sum(post_ref[...] * wrel_ref[...], axis=0, keepdims=True)
    logits = logits - bo
    mx = jnp.max(logits, axis=1, keepdims=True)
    e = jnp.exp(logits - mx)
    s = jnp.sum(e, axis=1, keepdims=True)
    att = e / s
    hq = jnp.dot(att, v_ref[...], preferred_element_type=jnp.float32)
    hid = jnp.maximum(
        jnp.dot(hq, wd1_ref[...], preferred_element_type=jnp.float32) + bd1_ref[...],
        0.0,
    )
    o_ref[...] = (
        jnp.dot(hid, wd2_ref[...], preferred_element_type=jnp.float32) + bd2_ref[...]
    )


def _attention(pos_query, Wq, bq, k_, v_, pos_obs_t, w_rel_col, Wd1, bd1, Wd2, bd2,
               out_dim):
    return pl.pallas_call(
        _attn_body,
        grid=(N_Q // _ATT_Q,),
        in_specs=[
            pl.BlockSpec((_ATT_Q, 3), lambda i: (i, 0)),
            pl.BlockSpec((3, LATENT), lambda i: (0, 0)),
            pl.BlockSpec((1, LATENT), lambda i: (0, 0)),
            pl.BlockSpec((N_OBS, LATENT), lambda i: (0, 0)),
            pl.BlockSpec((N_OBS, LATENT), lambda i: (0, 0)),
            pl.BlockSpec((3, N_OBS), lambda i: (0, 0)),
            pl.BlockSpec((3, 1), lambda i: (0, 0)),
            pl.BlockSpec((LATENT, LATENT), lambda i: (0, 0)),
            pl.BlockSpec((1, LATENT), lambda i: (0, 0)),
            pl.BlockSpec((LATENT, out_dim), lambda i: (0, 0)),
            pl.BlockSpec((1, out_dim), lambda i: (0, 0)),
        ],
        out_specs=pl.BlockSpec((_ATT_Q, out_dim), lambda i: (i, 0)),
        out_shape=jax.ShapeDtypeStruct((N_Q, out_dim), jnp.float32),
    )(pos_query, Wq, bq, k_, v_, pos_obs_t, w_rel_col, Wd1, bd1, Wd2, bd2)


_SC_PAD_ROWS = 10240
_SC_ROWS_W = 320
_SC_CH = 32
_SC_NCH = 10


def _make_sc_gather_mean():
    mesh = plsc.VectorSubcoreMesh(core_axis_name="c", subcore_axis_name="s")

    @functools.partial(
        pl.kernel,
        mesh=mesh,
        out_type=jax.ShapeDtypeStruct((_SC_PAD_ROWS, LATENT), jnp.float32),
        scratch_types=[
            pltpu.VMEM((_SC_NCH, _SC_CH * K), jnp.int32),
            pltpu.VMEM((_SC_CH * K, LATENT), jnp.float32),
            pltpu.VMEM((_SC_CH, LATENT), jnp.float32),
            pltpu.SemaphoreType.DMA,
        ],
    )
    def sc_gather_mean(h_hbm, idx_hbm, out_hbm, idx_v, rows_v, agg_v, sem):
        wid = lax.axis_index("c") * 16 + lax.axis_index("s")
        pltpu.sync_copy(idx_hbm.at[wid], idx_v)

        def chunk(c, carry):
            pltpu.async_copy(h_hbm.at[idx_v.at[c]], rows_v, sem).wait()

            def row(r, carry2):
                for vv in range(LATENT // 16):
                    sl = pl.ds(vv * 16, 16)
                    acc = (rows_v[4 * r, sl] + rows_v[4 * r + 1, sl]) + (
                        rows_v[4 * r + 2, sl] + rows_v[4 * r + 3, sl]
                    )
                    agg_v[r, sl] = acc * 0.25
                return carry2

            lax.fori_loop(0, _SC_CH, row, 0)
            pltpu.sync_copy(
                agg_v, out_hbm.at[pl.ds(wid * _SC_ROWS_W + c * _SC_CH, _SC_CH)]
            )
            return carry

        lax.fori_loop(0, _SC_NCH, chunk, 0)

    return sc_gather_mean


_sc_gather_mean_cached = functools.cache(_make_sc_gather_mean)


def _gather_mean(h, idx_mat):
    agg = _sc_gather_mean_cached()(h, idx_mat)
    return agg[:N_OBS]


def kernel(x_obs, pos_obs, pos_query, W_enc1, b_enc1, W_enc2, b_enc2,
           W_gs0, W_gn0, b_g0, W_gs1, W_gn1, b_g1,
           W_q, b_q, W_k, b_k, W_v, b_v, w_rel,
           W_dec1, b_dec1, W_dec2, b_dec2):
    out_dim = W_dec2.shape[1]
    b_enc1 = b_enc1.reshape(1, -1)
    b_enc2 = b_enc2.reshape(1, -1)
    b_g0 = b_g0.reshape(1, -1)
    b_g1 = b_g1.reshape(1, -1)
    b_q = b_q.reshape(1, -1)
    b_k = b_k.reshape(1, -1)
    b_v = b_v.reshape(1, -1)
    b_dec1 = b_dec1.reshape(1, -1)
    b_dec2 = b_dec2.reshape(1, -1)
    pos_obs_t = pos_obs.T
    w_rel_col = w_rel.reshape(-1, 1)
    Wkh, Wkp = W_k[:LATENT], W_k[LATENT:]
    Wvh, Wvp = W_v[:LATENT], W_v[LATENT:]

    h0 = _encode(x_obs, W_enc1, b_enc1, W_enc2, b_enc2)
    nbr = _knn(pos_obs, pos_obs_t)

    idx_flat = nbr.reshape(-1)
    idx_flat = jnp.pad(idx_flat, (0, _SC_PAD_ROWS * K - N_OBS * K))
    idx_mat = idx_flat.reshape(32, _SC_NCH, _SC_CH * K)

    agg0 = _gather_mean(h0, idx_mat)
    h1 = _gnn_layer(h0, agg0, W_gs0, W_gn0, b_g0)
    agg1 = _gather_mean(h1, idx_mat)
    h2 = _gnn_layer(h1, agg1, W_gs1, W_gn1, b_g1)

    k_, v_ = _kv(h2, pos_obs, Wkh, Wkp, b_k, Wvh, Wvp, b_v)
    out = _attention(pos_query, W_q, b_q, k_, v_, pos_obs_t, w_rel_col,
                     W_dec1, b_dec1, W_dec2, b_dec2, out_dim)
    return out
